# baseline (speedup 1.0000x reference)
import jax
import jax.numpy as jnp
from jax.experimental import pallas as pl
from jax.experimental.pallas import tpu as pltpu

_NCHUNK = 256


def _argmin_rows(key):
    k_rows = key.shape[0]
    val = key
    idx = jax.lax.broadcasted_iota(jnp.int32, key.shape, 0)
    while val.shape[0] > 8:
        h = val.shape[0] // 2
        lo_v, hi_v = val[:h], val[h:]
        lo_i, hi_i = idx[:h], idx[h:]
        take = hi_v < lo_v
        val = jnp.where(take, hi_v, lo_v)
        idx = jnp.where(take, hi_i, lo_i)
    m = jnp.min(val, axis=0, keepdims=True)
    cand = jnp.where(val == m, idx, k_rows)
    return jnp.min(cand, axis=0)


def _vq_kernel(z_ref, w_ref, out_ref, w2x_ref, w2c_ref):
    @pl.when(pl.program_id(0) == 0)
    def _():
        w = w_ref[...]
        w2x_ref[...] = -(w + w)
        w2c_ref[...] = jnp.sum(w * w, axis=1, keepdims=True)

    zt = z_ref[0]
    w2x = w2x_ref[...]
    w2c = w2c_ref[...]
    n = zt.shape[1]
    for c in range(0, n, _NCHUNK):
        nxw2 = jax.lax.dot_general(
            w2x, zt[:, c:c + _NCHUNK], (((1,), (0,)), ((), ())),
            preferred_element_type=jnp.float32,
            precision=jax.lax.Precision.DEFAULT)
        key = w2c + nxw2
        out_ref[0, 0, c:c + _NCHUNK] = _argmin_rows(key).astype(jnp.int32)


def kernel(z, W):
    t, a, b, c = z.shape
    n = b * c
    k = W.shape[0]
    z3 = z.reshape(t, a, n)
    out = pl.pallas_call(
        _vq_kernel,
        grid=(t,),
        in_specs=[
            pl.BlockSpec((1, a, n), lambda i: (i, 0, 0)),
            pl.BlockSpec((k, a), lambda i: (0, 0)),
        ],
        out_specs=pl.BlockSpec((1, 1, n), lambda i: (i, 0, 0)),
        out_shape=jax.ShapeDtypeStruct((t, 1, n), jnp.int32),
        scratch_shapes=[
            pltpu.VMEM((k, a), jnp.float32),
            pltpu.VMEM((k, 1), jnp.float32),
        ],
    )(z3, W)
    return out.reshape(t, b, c)


# speedup vs baseline: 1.0545x; 1.0545x over previous
import jax
import jax.numpy as jnp
from jax.experimental import pallas as pl
from jax.experimental.pallas import tpu as pltpu

_NCHUNK = 256


def _vq_kernel(z_ref, w_ref, out_ref, w2x_ref, w2c_ref):
    @pl.when(pl.program_id(0) == 0)
    def _():
        w = w_ref[...]
        w2x_ref[...] = -(w + w)
        w2c_ref[...] = jnp.sum(w * w, axis=1, keepdims=True)

    zt = z_ref[0]
    w2x = w2x_ref[...]
    w2c = w2c_ref[...]
    n = zt.shape[1]
    for c in range(0, n, _NCHUNK):
        nxw2 = jax.lax.dot_general(
            w2x, zt[:, c:c + _NCHUNK], (((1,), (0,)), ((), ())),
            preferred_element_type=jnp.float32,
            precision=jax.lax.Precision.DEFAULT)
        key = w2c + nxw2
        out_ref[0, 0, c:c + _NCHUNK] = jnp.argmin(key, axis=0).astype(jnp.int32)


def kernel(z, W):
    t, a, b, c = z.shape
    n = b * c
    k = W.shape[0]
    z3 = z.reshape(t, a, n)
    out = pl.pallas_call(
        _vq_kernel,
        grid=(t,),
        in_specs=[
            pl.BlockSpec((1, a, n), lambda i: (i, 0, 0)),
            pl.BlockSpec((k, a), lambda i: (0, 0)),
        ],
        out_specs=pl.BlockSpec((1, 1, n), lambda i: (i, 0, 0)),
        out_shape=jax.ShapeDtypeStruct((t, 1, n), jnp.int32),
        compiler_params=pltpu.CompilerParams(
            dimension_semantics=("arbitrary",)),
        scratch_shapes=[
            pltpu.VMEM((k, a), jnp.float32),
            pltpu.VMEM((k, 1), jnp.float32),
        ],
    )(z3, W)
    return out.reshape(t, b, c)


# speedup vs baseline: 1.1905x; 1.1290x over previous
import jax
import jax.numpy as jnp
from jax.experimental import pallas as pl
from jax.experimental.pallas import tpu as pltpu

_NCHUNK = 256
_TBLK = 2


def _vq_kernel(z_ref, w_ref, out_ref, w2x_ref, w2c_ref):
    @pl.when(pl.program_id(0) == 0)
    def _():
        w = w_ref[...]
        w2x_ref[...] = -(w + w)
        w2c_ref[...] = jnp.sum(w * w, axis=1, keepdims=True)

    w2x = w2x_ref[...]
    w2c = w2c_ref[...]
    n = z_ref.shape[2]
    for tt in range(z_ref.shape[0]):
        zt = z_ref[tt]
        for c in range(0, n, _NCHUNK):
            nxw2 = jax.lax.dot_general(
                w2x, zt[:, c:c + _NCHUNK], (((1,), (0,)), ((), ())),
                preferred_element_type=jnp.float32,
                precision=jax.lax.Precision.DEFAULT)
            key = w2c + nxw2
            out_ref[tt, 0, c:c + _NCHUNK] = jnp.argmin(key, axis=0).astype(jnp.int32)


def kernel(z, W):
    t, a, b, c = z.shape
    n = b * c
    k = W.shape[0]
    z3 = z.reshape(t, a, n)
    out = pl.pallas_call(
        _vq_kernel,
        grid=(t // _TBLK,),
        in_specs=[
            pl.BlockSpec((_TBLK, a, n), lambda i: (i, 0, 0)),
            pl.BlockSpec((k, a), lambda i: (0, 0)),
        ],
        out_specs=pl.BlockSpec((_TBLK, 1, n), lambda i: (i, 0, 0)),
        out_shape=jax.ShapeDtypeStruct((t, 1, n), jnp.int32),
        scratch_shapes=[
            pltpu.VMEM((k, a), jnp.float32),
            pltpu.VMEM((k, 1), jnp.float32),
        ],
    )(z3, W)
    return out.reshape(t, b, c)


# speedup vs baseline: 1.2029x; 1.0104x over previous
import jax
import jax.numpy as jnp
from jax.experimental import pallas as pl
from jax.experimental.pallas import tpu as pltpu

_NCHUNK = 256
_TBLK = 4


def _vq_kernel(z_ref, w_ref, out_ref, w2x_ref, w2c_ref):
    @pl.when(pl.program_id(0) == 0)
    def _():
        w = w_ref[...]
        w2x_ref[...] = -(w + w)
        w2c_ref[...] = jnp.sum(w * w, axis=1, keepdims=True)

    w2x = w2x_ref[...]
    w2c = w2c_ref[...]
    n = z_ref.shape[2]
    for tt in range(z_ref.shape[0]):
        zt = z_ref[tt]
        for c in range(0, n, _NCHUNK):
            nxw2 = jax.lax.dot_general(
                w2x, zt[:, c:c + _NCHUNK], (((1,), (0,)), ((), ())),
                preferred_element_type=jnp.float32,
                precision=jax.lax.Precision.DEFAULT)
            key = w2c + nxw2
            out_ref[tt, 0, c:c + _NCHUNK] = jnp.argmin(key, axis=0).astype(jnp.int32)


def kernel(z, W):
    t, a, b, c = z.shape
    n = b * c
    k = W.shape[0]
    z3 = z.reshape(t, a, n)
    out = pl.pallas_call(
        _vq_kernel,
        grid=(t // _TBLK,),
        in_specs=[
            pl.BlockSpec((_TBLK, a, n), lambda i: (i, 0, 0)),
            pl.BlockSpec((k, a), lambda i: (0, 0)),
        ],
        out_specs=pl.BlockSpec((_TBLK, 1, n), lambda i: (i, 0, 0)),
        out_shape=jax.ShapeDtypeStruct((t, 1, n), jnp.int32),
        scratch_shapes=[
            pltpu.VMEM((k, a), jnp.float32),
            pltpu.VMEM((k, 1), jnp.float32),
        ],
    )(z3, W)
    return out.reshape(t, b, c)
